# baseline (speedup 1.0000x reference)
import functools

import jax
import jax.numpy as jnp
from jax import lax
from jax.experimental import pallas as pl
from jax.experimental.pallas import tpu as pltpu
from jax.experimental.pallas import tpu_sc as plsc

N_ROWS = 100000
D = 128
BLOCK_ROWS = 25000
NUM_BLOCKS = N_ROWS // BLOCK_ROWS
N_IDX = 20000
NC = 2
NS = 16
NW = NC * NS
CHUNK = 128
SC0_CHUNKS = 4
SC1_CHUNKS = 6
SC0_PER_W = SC0_CHUNKS * CHUNK
SC1_PER_W = SC1_CHUNKS * CHUNK
SC0_TOTAL = NS * SC0_PER_W


def _tc_body(x_ref, y_ref, tok_ref, acc_ref):
    i = pl.program_id(0)

    @pl.when(i == 0)
    def _init():
        acc_ref[...] = jnp.zeros_like(acc_ref)

    xb = x_ref[...]
    y_ref[...] = xb
    acc_ref[...] += jnp.sum(xb.reshape(BLOCK_ROWS // 8, 8, D), axis=0)

    @pl.when(i == NUM_BLOCKS - 1)
    def _finish():
        tok_ref[...] = jnp.broadcast_to(
            jnp.sum(acc_ref[...], axis=0, keepdims=True) * (1.0 / N_ROWS),
            (CHUNK, D),
        )


_tc_copy_mean = pl.pallas_call(
    _tc_body,
    grid=(NUM_BLOCKS,),
    in_specs=[pl.BlockSpec((BLOCK_ROWS, D), lambda i: (i, 0))],
    out_specs=[
        pl.BlockSpec((BLOCK_ROWS, D), lambda i: (i, 0)),
        pl.BlockSpec((CHUNK, D), lambda i: (0, 0)),
    ],
    out_shape=[
        jax.ShapeDtypeStruct((N_ROWS, D), jnp.float32),
        jax.ShapeDtypeStruct((CHUNK, D), jnp.float32),
    ],
    scratch_shapes=[pltpu.VMEM((8, D), jnp.float32)],
)


_sc_mesh = plsc.VectorSubcoreMesh(core_axis_name="c", subcore_axis_name="s")


@functools.partial(
    pl.kernel,
    mesh=_sc_mesh,
    scratch_types=[
        pltpu.VMEM((SC1_CHUNKS, CHUNK), jnp.int32),
        pltpu.VMEM((CHUNK, D), jnp.float32),
        pltpu.SemaphoreType.DMA,
        pltpu.SemaphoreType.DMA,
    ],
)
def _sc_scatter(y_ref, tok_hbm, idx_hbm, idx_v, rows_v, isem, sem):
    c = lax.axis_index("c")
    s = lax.axis_index("s")
    base = jnp.where(
        c == 0,
        s * SC0_PER_W,
        jnp.minimum(SC0_TOTAL + s * SC1_PER_W, N_IDX - SC1_PER_W),
    )

    tok_copy = pltpu.async_copy(tok_hbm, rows_v, isem)

    def _run_chunks(js):
        idx_copies = [
            pltpu.async_copy(
                idx_hbm.at[pl.ds(base + j * CHUNK, CHUNK)], idx_v.at[j], isem
            )
            for j in js
        ]
        for ic in idx_copies:
            ic.wait()
        tok_copy.wait()
        copies = [
            pltpu.async_copy(rows_v, y_ref.at[idx_v.at[j]], sem) for j in js
        ]
        for sc in copies:
            sc.wait()

    @pl.when(c == 0)
    def _core0():
        _run_chunks(range(SC0_CHUNKS))

    @pl.when(c == 1)
    def _core1():
        _run_chunks(range(SC1_CHUNKS))


def kernel(x, idx_mask):
    idx = idx_mask.astype(jnp.int32)
    y, tok = _tc_copy_mean(x)
    y_ref = jax.new_ref(y)
    _sc_scatter(y_ref, tok, idx)
    return y_ref[...]


# speedup vs baseline: 1.0462x; 1.0462x over previous
import functools

import jax
import jax.numpy as jnp
from jax import lax
from jax.experimental import pallas as pl
from jax.experimental.pallas import tpu as pltpu
from jax.experimental.pallas import tpu_sc as plsc

N_ROWS = 100000
D = 128
BLOCK_ROWS = 25000
NUM_BLOCKS = N_ROWS // BLOCK_ROWS
ACC_ROWS = 40
N_IDX = 20000
NC = 2
NS = 16
NW = NC * NS
CHUNK = 128
SC0_CHUNKS = 4
SC1_CHUNKS = 6
SC0_PER_W = SC0_CHUNKS * CHUNK
SC1_PER_W = SC1_CHUNKS * CHUNK
SC0_TOTAL = NS * SC0_PER_W


def _tc_body(x_ref, y_ref, tok_ref, acc_ref):
    i = pl.program_id(0)

    @pl.when(i == 0)
    def _init():
        acc_ref[...] = jnp.zeros_like(acc_ref)

    xb = x_ref[...]
    y_ref[...] = xb
    acc_ref[...] += jnp.sum(xb.reshape(BLOCK_ROWS // ACC_ROWS, ACC_ROWS, D), axis=0)

    @pl.when(i == NUM_BLOCKS - 1)
    def _finish():
        tok_ref[...] = jnp.broadcast_to(
            jnp.sum(acc_ref[...], axis=0, keepdims=True) * (1.0 / N_ROWS),
            (CHUNK, D),
        )


_tc_copy_mean = pl.pallas_call(
    _tc_body,
    grid=(NUM_BLOCKS,),
    in_specs=[pl.BlockSpec((BLOCK_ROWS, D), lambda i: (i, 0))],
    out_specs=[
        pl.BlockSpec((BLOCK_ROWS, D), lambda i: (i, 0)),
        pl.BlockSpec((CHUNK, D), lambda i: (0, 0)),
    ],
    out_shape=[
        jax.ShapeDtypeStruct((N_ROWS, D), jnp.float32),
        jax.ShapeDtypeStruct((CHUNK, D), jnp.float32),
    ],
    scratch_shapes=[pltpu.VMEM((ACC_ROWS, D), jnp.float32)],
)


_sc_mesh = plsc.VectorSubcoreMesh(core_axis_name="c", subcore_axis_name="s")


@functools.partial(
    pl.kernel,
    mesh=_sc_mesh,
    scratch_types=[
        pltpu.VMEM((SC1_CHUNKS, CHUNK), jnp.int32),
        pltpu.VMEM((CHUNK, D), jnp.float32),
        pltpu.SemaphoreType.DMA,
        pltpu.SemaphoreType.DMA,
    ],
)
def _sc_scatter(y_ref, tok_hbm, idx_hbm, idx_v, rows_v, isem, sem):
    c = lax.axis_index("c")
    s = lax.axis_index("s")
    base = jnp.where(
        c == 0,
        s * SC0_PER_W,
        jnp.minimum(SC0_TOTAL + s * SC1_PER_W, N_IDX - SC1_PER_W),
    )

    tok_copy = pltpu.async_copy(tok_hbm, rows_v, isem)

    def _run_chunks(js):
        idx_copies = [
            pltpu.async_copy(
                idx_hbm.at[pl.ds(base + j * CHUNK, CHUNK)], idx_v.at[j], isem
            )
            for j in js
        ]
        for ic in idx_copies:
            ic.wait()
        tok_copy.wait()
        copies = [
            pltpu.async_copy(rows_v, y_ref.at[idx_v.at[j]], sem) for j in js
        ]
        for sc in copies:
            sc.wait()

    @pl.when(c == 0)
    def _core0():
        _run_chunks(range(SC0_CHUNKS))

    @pl.when(c == 1)
    def _core1():
        _run_chunks(range(SC1_CHUNKS))


def kernel(x, idx_mask):
    idx = idx_mask.astype(jnp.int32)
    y, tok = _tc_copy_mean(x)
    y_ref = jax.new_ref(y)
    _sc_scatter(y_ref, tok, idx)
    return y_ref[...]


# speedup vs baseline: 1.0580x; 1.0113x over previous
import functools

import jax
import jax.numpy as jnp
from jax import lax
from jax.experimental import pallas as pl
from jax.experimental.pallas import tpu as pltpu
from jax.experimental.pallas import tpu_sc as plsc

N_ROWS = 100000
D = 128
BLOCK_ROWS = 25000
NUM_BLOCKS = N_ROWS // BLOCK_ROWS
ACC_ROWS = 200
N_IDX = 20000
NC = 2
NS = 16
NW = NC * NS
CHUNK = 128
SC0_CHUNKS = 4
SC1_CHUNKS = 6
SC0_PER_W = SC0_CHUNKS * CHUNK
SC1_PER_W = SC1_CHUNKS * CHUNK
SC0_TOTAL = NS * SC0_PER_W


def _tc_body(x_ref, y_ref, tok_ref, acc_ref):
    i = pl.program_id(0)

    @pl.when(i == 0)
    def _init():
        acc_ref[...] = jnp.zeros_like(acc_ref)

    xb = x_ref[...]
    y_ref[...] = xb
    acc_ref[...] += jnp.sum(xb.reshape(BLOCK_ROWS // ACC_ROWS, ACC_ROWS, D), axis=0)

    @pl.when(i == NUM_BLOCKS - 1)
    def _finish():
        tok_ref[...] = jnp.broadcast_to(
            jnp.sum(acc_ref[...], axis=0, keepdims=True) * (1.0 / N_ROWS),
            (CHUNK, D),
        )


_tc_copy_mean = pl.pallas_call(
    _tc_body,
    grid=(NUM_BLOCKS,),
    in_specs=[pl.BlockSpec((BLOCK_ROWS, D), lambda i: (i, 0))],
    out_specs=[
        pl.BlockSpec((BLOCK_ROWS, D), lambda i: (i, 0)),
        pl.BlockSpec((CHUNK, D), lambda i: (0, 0)),
    ],
    out_shape=[
        jax.ShapeDtypeStruct((N_ROWS, D), jnp.float32),
        jax.ShapeDtypeStruct((CHUNK, D), jnp.float32),
    ],
    scratch_shapes=[pltpu.VMEM((ACC_ROWS, D), jnp.float32)],
)


_sc_mesh = plsc.VectorSubcoreMesh(core_axis_name="c", subcore_axis_name="s")


@functools.partial(
    pl.kernel,
    mesh=_sc_mesh,
    scratch_types=[
        pltpu.VMEM((SC1_CHUNKS, CHUNK), jnp.int32),
        pltpu.VMEM((CHUNK, D), jnp.float32),
        pltpu.SemaphoreType.DMA,
        pltpu.SemaphoreType.DMA,
    ],
)
def _sc_scatter(y_ref, tok_hbm, idx_hbm, idx_v, rows_v, isem, sem):
    c = lax.axis_index("c")
    s = lax.axis_index("s")
    base = jnp.where(
        c == 0,
        s * SC0_PER_W,
        jnp.minimum(SC0_TOTAL + s * SC1_PER_W, N_IDX - SC1_PER_W),
    )

    tok_copy = pltpu.async_copy(tok_hbm, rows_v, isem)

    def _run_chunks(js):
        idx_copies = [
            pltpu.async_copy(
                idx_hbm.at[pl.ds(base + j * CHUNK, CHUNK)], idx_v.at[j], isem
            )
            for j in js
        ]
        for ic in idx_copies:
            ic.wait()
        tok_copy.wait()
        copies = [
            pltpu.async_copy(rows_v, y_ref.at[idx_v.at[j]], sem) for j in js
        ]
        for sc in copies:
            sc.wait()

    @pl.when(c == 0)
    def _core0():
        _run_chunks(range(SC0_CHUNKS))

    @pl.when(c == 1)
    def _core1():
        _run_chunks(range(SC1_CHUNKS))


def kernel(x, idx_mask):
    idx = idx_mask.astype(jnp.int32)
    y, tok = _tc_copy_mean(x)
    y_ref = jax.new_ref(y)
    _sc_scatter(y_ref, tok, idx)
    return y_ref[...]
